# baseline (speedup 1.0000x reference)
import jax
import jax.numpy as jnp
from jax.experimental import pallas as pl

_PACK = 4
_BLOCK_ROWS = 2000


def _pointwise_mm_block(x_ref, w_ref, b_ref, o_ref):
    o_ref[...] = (
        jnp.dot(x_ref[...], w_ref[...], preferred_element_type=jnp.float32)
        + b_ref[...]
    )


def kernel(input, kernel, bias):
    n, c_in = input.shape
    c_out = kernel.shape[1]
    packed_n = n // _PACK
    x = input.reshape(packed_n, _PACK * c_in)

    w_big = jnp.zeros((_PACK * c_in, _PACK * c_out), dtype=kernel.dtype)
    for i in range(_PACK):
        w_big = w_big.at[i * c_in:(i + 1) * c_in,
                         i * c_out:(i + 1) * c_out].set(kernel)
    b_big = jnp.tile(bias, (1, _PACK))

    out = pl.pallas_call(
        _pointwise_mm_block,
        grid=(packed_n // _BLOCK_ROWS,),
        in_specs=[
            pl.BlockSpec((_BLOCK_ROWS, _PACK * c_in), lambda i: (i, 0)),
            pl.BlockSpec((_PACK * c_in, _PACK * c_out), lambda i: (0, 0)),
            pl.BlockSpec((1, _PACK * c_out), lambda i: (0, 0)),
        ],
        out_specs=pl.BlockSpec((_BLOCK_ROWS, _PACK * c_out), lambda i: (i, 0)),
        out_shape=jax.ShapeDtypeStruct((packed_n, _PACK * c_out), jnp.float32),
    )(x, w_big, b_big)
    return out.reshape(n, c_out)


# speedup vs baseline: 1.2729x; 1.2729x over previous
import jax
import jax.numpy as jnp
from jax.experimental import pallas as pl

_BLOCK_ROWS = 8000


def _pointwise_mm_block(x_ref, w_ref, b_ref, o_ref):
    o_ref[...] = (
        jnp.dot(x_ref[...], w_ref[...], preferred_element_type=jnp.float32)
        + b_ref[...]
    )


def kernel(input, kernel, bias):
    n, c_in = input.shape
    c_out = kernel.shape[1]
    out = pl.pallas_call(
        _pointwise_mm_block,
        grid=(n // _BLOCK_ROWS,),
        in_specs=[
            pl.BlockSpec((_BLOCK_ROWS, c_in), lambda i: (i, 0)),
            pl.BlockSpec((c_in, c_out), lambda i: (0, 0)),
            pl.BlockSpec((1, c_out), lambda i: (0, 0)),
        ],
        out_specs=pl.BlockSpec((_BLOCK_ROWS, c_out), lambda i: (i, 0)),
        out_shape=jax.ShapeDtypeStruct((n, c_out), jnp.float32),
    )(input, kernel, bias)
    return out


# speedup vs baseline: 14.1210x; 11.0938x over previous
import jax
import jax.numpy as jnp
from jax.experimental import pallas as pl

_BLOCK_COLS = 65536


def _pointwise_mm_block(xt_ref, w_ref, bt_ref, ot_ref):
    ot_ref[...] = (
        jax.lax.dot_general(
            w_ref[...], xt_ref[...],
            dimension_numbers=(((0,), (0,)), ((), ())),
            preferred_element_type=jnp.float32,
        )
        + bt_ref[...]
    )


def kernel(input, kernel, bias):
    n, c_in = input.shape
    c_out = kernel.shape[1]
    xt = input.T
    bt = bias.T
    grid = (pl.cdiv(n, _BLOCK_COLS),)
    out_t = pl.pallas_call(
        _pointwise_mm_block,
        grid=grid,
        in_specs=[
            pl.BlockSpec((c_in, _BLOCK_COLS), lambda i: (0, i)),
            pl.BlockSpec((c_in, c_out), lambda i: (0, 0)),
            pl.BlockSpec((c_out, 1), lambda i: (0, 0)),
        ],
        out_specs=pl.BlockSpec((c_out, _BLOCK_COLS), lambda i: (0, i)),
        out_shape=jax.ShapeDtypeStruct((c_out, n), jnp.float32),
    )(xt, kernel, bt)
    return out_t.T
